# Initial kernel scaffold; baseline (speedup 1.0000x reference)
import jax
import jax.numpy as jnp
from jax.experimental import pallas as pl


def kernel(pos, batch, c1_W1, c1_b1, c1_g1, c1_be1, c1_W2, c1_b2, c1_g2, c1_be2, c1_W3, c1_b3, c2_W, c2_b, l1_W, l1_b, m1_W, m1_b, m2_W, m2_b, h_W, h_b):
    raise NotImplementedError("write your pallas kernel here")


# speedup vs baseline: 12.7943x; 12.7943x over previous
import jax
import jax.numpy as jnp
from jax.experimental import pallas as pl
from jax.experimental.pallas import tpu as pltpu
from jax.experimental.pallas import tpu_sc as plsc

_N = 8192
_K = 20
_G = 16
_R = 256
_NB = _N // _R
_NEG = -jnp.inf
_KP = 32
_W_FAST = 2048


def _make_knn_body(window):
    def body(starts_ref, x_ref, xT_ref, brow_ref, bcol_ref, idx_ref):
        i = pl.program_id(0)
        s = starts_ref[i]
        xw = x_ref[pl.ds(s, window), :]
        xtr = xT_ref[...]
        mm = jax.lax.dot_general(xw, xtr, (((1,), (0,)), ((), ())),
                                 preferred_element_type=jnp.float32)
        cn = jnp.sum(xw * xw, axis=1, keepdims=True)
        rn = jnp.sum(xtr * xtr, axis=0, keepdims=True)
        dT = cn + rn - 2.0 * mm
        candio = s + jax.lax.broadcasted_iota(jnp.int32, dT.shape, 0)
        rowio = i * _R + jax.lax.broadcasted_iota(jnp.int32, dT.shape, 1)
        bw = bcol_ref[pl.ds(s, window), :]
        valid = (bw == brow_ref[...]) & (candio != rowio)
        dT = jnp.where(valid, dT, jnp.inf)

        def step(t, d):
            m = jnp.min(d, axis=0, keepdims=True)
            cand = jnp.where(d <= m, candio, jnp.int32(2147483647))
            j = jnp.min(cand, axis=0, keepdims=True)
            idx_ref[pl.ds(t, 1), :] = j
            return jnp.where(candio == j, jnp.inf, d)

        jax.lax.fori_loop(0, _K, step, dT)

    return body


def _knn_call(x, xT, brow, bcol, starts, window):
    n, f = x.shape
    grid_spec = pltpu.PrefetchScalarGridSpec(
        num_scalar_prefetch=1,
        grid=(_NB,),
        in_specs=[
            pl.BlockSpec((n, f), lambda i, *_: (0, 0)),
            pl.BlockSpec((f, _R), lambda i, *_: (0, i)),
            pl.BlockSpec((1, _R), lambda i, *_: (0, i)),
            pl.BlockSpec((n, 1), lambda i, *_: (0, 0)),
        ],
        out_specs=pl.BlockSpec((_KP, _R), lambda i, *_: (0, i)),
    )
    return pl.pallas_call(
        _make_knn_body(window),
        grid_spec=grid_spec,
        out_shape=jax.ShapeDtypeStruct((_KP, n), jnp.int32),
        compiler_params=pltpu.CompilerParams(
            dimension_semantics=("arbitrary",)),
    )(starts, x, xT, brow, bcol)


def _knn(x, batchrow, batchcol, blk_start, blk_end):
    n = x.shape[0]
    xT = x.T
    s_fast = jnp.minimum(blk_start, n - _W_FAST).astype(jnp.int32)
    fits = jnp.all(blk_end - s_fast <= _W_FAST)
    zeros = jnp.zeros_like(s_fast)

    def fast(xx, xxT):
        return _knn_call(xx, xxT, batchrow, batchcol, s_fast, _W_FAST)

    def slow(xx, xxT):
        return _knn_call(xx, xxT, batchrow, batchcol, zeros, n)

    idxp = jax.lax.cond(fits, fast, slow, x, xT)
    return idxp[:_K]


def _sc_gather(values, idx_flat, window=128):
    num_idx = idx_flat.shape[1]
    dv = values.shape[1]
    mesh = plsc.VectorSubcoreMesh(core_axis_name="c", subcore_axis_name="s")

    @pl.kernel(out_type=jax.ShapeDtypeStruct((num_idx, dv), values.dtype),
               mesh=mesh)
    def k(x_hbm, i_hbm, o_hbm):
        def body(i_vmem, o_vmem):
            pltpu.sync_copy(x_hbm.at[i_vmem.at[0]], o_vmem)

        pltpu.emit_pipeline(
            body,
            grid=(num_idx // window,),
            in_specs=[pl.BlockSpec((1, window), index_map=lambda i: (0, i))],
            out_specs=[pl.BlockSpec((window, dv), index_map=lambda i: (i, 0))],
            core_axis_name=("c", "s"),
            dimension_semantics=(pltpu.PARALLEL,),
        )(i_hbm, o_hbm)

    return k(values, idx_flat)


def _edge1(xi_ref, p_ref, w_ref, b_ref, j):
    xi = xi_ref[...]
    xj = p_ref[j][:, :3]
    e = jnp.concatenate([xi, xj - xi], axis=1)
    return jax.lax.dot_general(e, w_ref[...], (((1,), (0,)), ((), ())),
                               preferred_element_type=jnp.float32) + b_ref[...]


def _stats_body(xi_ref, p_ref, w_ref, b_ref, s_ref):
    i = pl.program_id(0)

    @pl.when(i == 0)
    def _():
        s_ref[...] = jnp.zeros_like(s_ref)

    dd = w_ref.shape[1]
    s1 = jnp.zeros((1, dd), jnp.float32)
    s2 = jnp.zeros((1, dd), jnp.float32)
    for j in range(_K):
        z = _edge1(xi_ref, p_ref, w_ref, b_ref, j)
        s1 = s1 + jnp.sum(z, axis=0, keepdims=True)
        s2 = s2 + jnp.sum(z * z, axis=0, keepdims=True)
    s_ref[0:1, :] += s1
    s_ref[1:2, :] += s2


def _stats(x, P, w, b):
    dd = w.shape[1]
    return pl.pallas_call(
        _stats_body,
        grid=(_NB,),
        in_specs=[
            pl.BlockSpec((_R, 3), lambda i: (i, 0)),
            pl.BlockSpec((_K, _R, 128), lambda i: (0, i, 0)),
            pl.BlockSpec((6, dd), lambda i: (0, 0)),
            pl.BlockSpec((1, dd), lambda i: (0, 0)),
        ],
        out_specs=pl.BlockSpec((8, dd), lambda i: (0, 0)),
        out_shape=jax.ShapeDtypeStruct((8, dd), jnp.float32),
        compiler_params=pltpu.CompilerParams(
            dimension_semantics=("arbitrary",)),
    )(x, P, w, b)


def _mlp1_body(xi_ref, p_ref, w1_ref, b1_ref, m_ref, sd_ref, g_ref, be_ref,
               w2_ref, b2_ref, z2_ref, s_ref):
    i = pl.program_id(0)

    @pl.when(i == 0)
    def _():
        s_ref[...] = jnp.zeros_like(s_ref)

    dd = w2_ref.shape[1]
    bm = m_ref[...]
    sd = sd_ref[...]
    g = g_ref[...]
    be = be_ref[...]
    w2 = w2_ref[...]
    b2 = b2_ref[...]
    s1 = jnp.zeros((1, dd), jnp.float32)
    s2 = jnp.zeros((1, dd), jnp.float32)
    for j in range(_K):
        z = _edge1(xi_ref, p_ref, w1_ref, b1_ref, j)
        h = jnp.maximum((z - bm) / sd * g + be, 0.0)
        z2 = jax.lax.dot_general(h, w2, (((1,), (0,)), ((), ())),
                                 preferred_element_type=jnp.float32) + b2
        z2_ref[j] = z2
        s1 = s1 + jnp.sum(z2, axis=0, keepdims=True)
        s2 = s2 + jnp.sum(z2 * z2, axis=0, keepdims=True)
    s_ref[0:1, :] += s1
    s_ref[1:2, :] += s2


def _mlp1(x, P, w1, b1, bm, sd, g, be, w2, b2):
    dd = w2.shape[1]
    return pl.pallas_call(
        _mlp1_body,
        grid=(_NB,),
        in_specs=[
            pl.BlockSpec((_R, 3), lambda i: (i, 0)),
            pl.BlockSpec((_K, _R, 128), lambda i: (0, i, 0)),
            pl.BlockSpec((6, dd), lambda i: (0, 0)),
            pl.BlockSpec((1, dd), lambda i: (0, 0)),
            pl.BlockSpec((1, dd), lambda i: (0, 0)),
            pl.BlockSpec((1, dd), lambda i: (0, 0)),
            pl.BlockSpec((1, dd), lambda i: (0, 0)),
            pl.BlockSpec((1, dd), lambda i: (0, 0)),
            pl.BlockSpec((dd, dd), lambda i: (0, 0)),
            pl.BlockSpec((1, dd), lambda i: (0, 0)),
        ],
        out_specs=[
            pl.BlockSpec((_K, _R, dd), lambda i: (0, i, 0)),
            pl.BlockSpec((8, dd), lambda i: (0, 0)),
        ],
        out_shape=[
            jax.ShapeDtypeStruct((_K, _N, dd), jnp.float32),
            jax.ShapeDtypeStruct((8, dd), jnp.float32),
        ],
        compiler_params=pltpu.CompilerParams(
            dimension_semantics=("arbitrary",)),
    )(x, P, w1, b1, bm, sd, g, be, w2, b2)


def _mlp2max_body(z2_ref, m_ref, sd_ref, g_ref, be_ref, w_ref, b_ref, x1_ref):
    bm = m_ref[...]
    sd = sd_ref[...]
    g = g_ref[...]
    be = be_ref[...]
    w = w_ref[...]
    b = b_ref[...]
    acc = None
    for j in range(_K):
        h = jnp.maximum((z2_ref[j] - bm) / sd * g + be, 0.0)
        z3 = jax.lax.dot_general(h, w, (((1,), (0,)), ((), ())),
                                 preferred_element_type=jnp.float32) + b
        acc = z3 if acc is None else jnp.maximum(acc, z3)
    x1_ref[...] = jnp.concatenate(
        [acc, jnp.zeros((acc.shape[0], 128 - acc.shape[1]), jnp.float32)],
        axis=1)


def _mlp2max(z2, bm, sd, g, be, w, b):
    dd = w.shape[1]
    return pl.pallas_call(
        _mlp2max_body,
        grid=(_NB,),
        in_specs=[
            pl.BlockSpec((_K, _R, dd), lambda i: (0, i, 0)),
            pl.BlockSpec((1, dd), lambda i: (0, 0)),
            pl.BlockSpec((1, dd), lambda i: (0, 0)),
            pl.BlockSpec((1, dd), lambda i: (0, 0)),
            pl.BlockSpec((1, dd), lambda i: (0, 0)),
            pl.BlockSpec((dd, dd), lambda i: (0, 0)),
            pl.BlockSpec((1, dd), lambda i: (0, 0)),
        ],
        out_specs=pl.BlockSpec((_R, 128), lambda i: (i, 0)),
        out_shape=jax.ShapeDtypeStruct((_N, 128), jnp.float32),
        compiler_params=pltpu.CompilerParams(
            dimension_semantics=("arbitrary",)),
    )(z2, bm, sd, g, be, w, b)


def _l1segmax_body(x1_ref, xg_ref, wc_ref, bc_ref, bcol_ref, w_ref, b_ref,
                   seg_ref):
    i = pl.program_id(0)

    @pl.when(i == 0)
    def _():
        seg_ref[...] = jnp.full_like(seg_ref, _NEG)

    xi = x1_ref[...][:, :64]
    wc = wc_ref[...]
    bc = bc_ref[...]
    acc = None
    for j in range(_K):
        e2 = jnp.concatenate([xi, xg_ref[j][:, :64] - xi], axis=1)
        z = jax.lax.dot_general(e2, wc, (((1,), (0,)), ((), ())),
                                preferred_element_type=jnp.float32) + bc
        acc = z if acc is None else jnp.maximum(acc, z)
    e = jnp.concatenate([xi, acc], axis=1)
    o = jax.lax.dot_general(e, w_ref[...], (((1,), (0,)), ((), ())),
                            preferred_element_type=jnp.float32) + b_ref[...]
    bb = bcol_ref[...]
    for s in range(_G):
        vals = jnp.max(jnp.where(bb == s, o, _NEG), axis=0, keepdims=True)
        seg_ref[s:s + 1, :] = jnp.maximum(seg_ref[s:s + 1, :], vals)


def _l1segmax(x1, Xg, wc, bc, batchcol, w, b):
    do = w.shape[1]
    return pl.pallas_call(
        _l1segmax_body,
        grid=(_NB,),
        in_specs=[
            pl.BlockSpec((_R, 128), lambda i: (i, 0)),
            pl.BlockSpec((_K, _R, 128), lambda i: (0, i, 0)),
            pl.BlockSpec((128, 128), lambda i: (0, 0)),
            pl.BlockSpec((1, 128), lambda i: (0, 0)),
            pl.BlockSpec((_R, 1), lambda i: (i, 0)),
            pl.BlockSpec((192, do), lambda i: (0, 0)),
            pl.BlockSpec((1, do), lambda i: (0, 0)),
        ],
        out_specs=pl.BlockSpec((_G, do), lambda i: (0, 0)),
        out_shape=jax.ShapeDtypeStruct((_G, do), jnp.float32),
        compiler_params=pltpu.CompilerParams(
            dimension_semantics=("arbitrary",)),
    )(x1, Xg, wc, bc, batchcol, w, b)


def _head_body(x_ref, w1_ref, b1_ref, w2_ref, b2_ref, w3_ref, b3_ref, o_ref):
    a = jax.lax.dot_general(x_ref[...], w1_ref[...], (((1,), (0,)), ((), ())),
                            preferred_element_type=jnp.float32) + b1_ref[...]
    a = jax.lax.dot_general(a, w2_ref[...], (((1,), (0,)), ((), ())),
                            preferred_element_type=jnp.float32) + b2_ref[...]
    o_ref[...] = jax.lax.dot_general(a, w3_ref[...], (((1,), (0,)), ((), ())),
                                     preferred_element_type=jnp.float32) + b3_ref[...]


def _head(x, w1, b1, w2, b2, w3, b3):
    return pl.pallas_call(
        _head_body,
        out_shape=jax.ShapeDtypeStruct((x.shape[0], w3.shape[1]), jnp.float32),
    )(x, w1, b1, w2, b2, w3, b3)


def kernel(pos, batch, c1_W1, c1_b1, c1_g1, c1_be1, c1_W2, c1_b2, c1_g2,
           c1_be2, c1_W3, c1_b3, c2_W, c2_b, l1_W, l1_b, m1_W, m1_b, m2_W,
           m2_b, h_W, h_b):
    batch = batch.astype(jnp.int32)
    brow = batch.reshape(1, _N)
    bcol = batch.reshape(_N, 1)
    nedges = float(_N * _K)
    seg_lo = jnp.searchsorted(batch, jnp.arange(_G, dtype=batch.dtype),
                              side="left").astype(jnp.int32)
    seg_hi = jnp.searchsorted(batch, jnp.arange(_G, dtype=batch.dtype),
                              side="right").astype(jnp.int32)
    b2d = batch.reshape(_NB, _R)
    blk_start = seg_lo[b2d[:, 0]]
    blk_end = seg_hi[b2d[:, -1]]

    def bn_params(sums):
        m = sums[0] / nedges
        var = sums[1] / nedges - m * m
        sd = jnp.sqrt(var + 1e-5)
        return m.reshape(1, -1), sd.reshape(1, -1)

    idx1 = _knn(pos, brow, bcol, blk_start, blk_end)
    posp = jnp.concatenate([pos, jnp.zeros((_N, 125), jnp.float32)], axis=1)
    P1 = _sc_gather(posp, idx1.reshape(1, -1)).reshape(_K, _N, 128)
    sums1 = _stats(pos, P1, c1_W1, c1_b1.reshape(1, -1))
    m1, sd1 = bn_params(sums1)
    z2, sums2 = _mlp1(pos, P1, c1_W1, c1_b1.reshape(1, -1), m1, sd1,
                      c1_g1.reshape(1, -1), c1_be1.reshape(1, -1),
                      c1_W2, c1_b2.reshape(1, -1))
    m2, sd2 = bn_params(sums2)
    x1 = _mlp2max(z2, m2, sd2, c1_g2.reshape(1, -1), c1_be2.reshape(1, -1),
                  c1_W3, c1_b3.reshape(1, -1))

    idx2 = _knn(x1, brow, bcol, blk_start, blk_end)
    Xg = _sc_gather(x1, idx2.reshape(1, -1)).reshape(_K, _N, 128)

    seg = _l1segmax(x1, Xg, c2_W, c2_b.reshape(1, -1), bcol, l1_W,
                    l1_b.reshape(1, -1))
    return _head(seg, m1_W, m1_b.reshape(1, -1), m2_W, m2_b.reshape(1, -1),
                 h_W, h_b.reshape(1, -1))


# speedup vs baseline: 15.6879x; 1.2262x over previous
import jax
import jax.numpy as jnp
from jax.experimental import pallas as pl
from jax.experimental.pallas import tpu as pltpu
from jax.experimental.pallas import tpu_sc as plsc

_N = 8192
_K = 20
_G = 16
_R = 256
_NB = _N // _R
_NEG = -jnp.inf
_KP = 32
_W_FAST = 1536


def _make_knn_body(window):
    def body(starts_ref, x_ref, xT_ref, brow_ref, bcol_ref, idx_ref):
        i = pl.program_id(0)
        s = starts_ref[i]
        xw = x_ref[pl.ds(s, window), :]
        xtr = xT_ref[...]
        mm = jax.lax.dot_general(xw, xtr, (((1,), (0,)), ((), ())),
                                 preferred_element_type=jnp.float32)
        cn = jnp.sum(xw * xw, axis=1, keepdims=True)
        rn = jnp.sum(xtr * xtr, axis=0, keepdims=True)
        dT = cn + rn - 2.0 * mm
        candio = s + jax.lax.broadcasted_iota(jnp.int32, dT.shape, 0)
        rowio = i * _R + jax.lax.broadcasted_iota(jnp.int32, dT.shape, 1)
        bw = bcol_ref[pl.ds(s, window), :]
        valid = (bw == brow_ref[...]) & (candio != rowio)
        dT = jnp.where(valid, dT, jnp.inf)

        def step(t, d):
            m = jnp.min(d, axis=0, keepdims=True)
            cand = jnp.where(d <= m, candio, jnp.int32(2147483647))
            j = jnp.min(cand, axis=0, keepdims=True)
            idx_ref[pl.ds(t, 1), :] = j
            return jnp.where(candio == j, jnp.inf, d)

        jax.lax.fori_loop(0, _K, step, dT)

    return body


def _knn_call(x, xT, brow, bcol, starts, window):
    n, f = x.shape
    grid_spec = pltpu.PrefetchScalarGridSpec(
        num_scalar_prefetch=1,
        grid=(_NB,),
        in_specs=[
            pl.BlockSpec((n, f), lambda i, *_: (0, 0)),
            pl.BlockSpec((f, _R), lambda i, *_: (0, i)),
            pl.BlockSpec((1, _R), lambda i, *_: (0, i)),
            pl.BlockSpec((n, 1), lambda i, *_: (0, 0)),
        ],
        out_specs=pl.BlockSpec((_KP, _R), lambda i, *_: (0, i)),
    )
    return pl.pallas_call(
        _make_knn_body(window),
        grid_spec=grid_spec,
        out_shape=jax.ShapeDtypeStruct((_KP, n), jnp.int32),
        compiler_params=pltpu.CompilerParams(
            dimension_semantics=("arbitrary",)),
    )(starts, x, xT, brow, bcol)


def _knn(x, batchrow, batchcol, blk_start, blk_end):
    n = x.shape[0]
    xT = x.T
    s_fast = jnp.minimum(blk_start, n - _W_FAST).astype(jnp.int32)
    fits = jnp.all(blk_end - s_fast <= _W_FAST)
    zeros = jnp.zeros_like(s_fast)

    def fast(xx, xxT):
        return _knn_call(xx, xxT, batchrow, batchcol, s_fast, _W_FAST)

    def slow(xx, xxT):
        return _knn_call(xx, xxT, batchrow, batchcol, zeros, n)

    idxp = jax.lax.cond(fits, fast, slow, x, xT)
    return idxp[:_K]


def _sc_gather(values, idx_flat, window=128):
    num_idx = idx_flat.shape[1]
    dv = values.shape[1]
    mesh = plsc.VectorSubcoreMesh(core_axis_name="c", subcore_axis_name="s")

    @pl.kernel(out_type=jax.ShapeDtypeStruct((num_idx, dv), values.dtype),
               mesh=mesh)
    def k(x_hbm, i_hbm, o_hbm):
        def body(i_vmem, o_vmem):
            pltpu.sync_copy(x_hbm.at[i_vmem.at[0]], o_vmem)

        pltpu.emit_pipeline(
            body,
            grid=(num_idx // window,),
            in_specs=[pl.BlockSpec((1, window), index_map=lambda i: (0, i))],
            out_specs=[pl.BlockSpec((window, dv), index_map=lambda i: (i, 0))],
            core_axis_name=("c", "s"),
            dimension_semantics=(pltpu.PARALLEL,),
        )(i_hbm, o_hbm)

    return k(values, idx_flat)


def _edge1(xi_ref, p_ref, w_ref, b_ref, j):
    xi = xi_ref[...]
    xj = p_ref[j][:, :3]
    e = jnp.concatenate([xi, xj - xi], axis=1)
    return jax.lax.dot_general(e, w_ref[...], (((1,), (0,)), ((), ())),
                               preferred_element_type=jnp.float32) + b_ref[...]


def _stats_body(xi_ref, p_ref, w_ref, b_ref, s_ref):
    i = pl.program_id(0)

    @pl.when(i == 0)
    def _():
        s_ref[...] = jnp.zeros_like(s_ref)

    dd = w_ref.shape[1]
    s1 = jnp.zeros((1, dd), jnp.float32)
    s2 = jnp.zeros((1, dd), jnp.float32)
    for j in range(_K):
        z = _edge1(xi_ref, p_ref, w_ref, b_ref, j)
        s1 = s1 + jnp.sum(z, axis=0, keepdims=True)
        s2 = s2 + jnp.sum(z * z, axis=0, keepdims=True)
    s_ref[0:1, :] += s1
    s_ref[1:2, :] += s2


def _stats(x, P, w, b):
    dd = w.shape[1]
    return pl.pallas_call(
        _stats_body,
        grid=(_NB,),
        in_specs=[
            pl.BlockSpec((_R, 3), lambda i: (i, 0)),
            pl.BlockSpec((_K, _R, 128), lambda i: (0, i, 0)),
            pl.BlockSpec((6, dd), lambda i: (0, 0)),
            pl.BlockSpec((1, dd), lambda i: (0, 0)),
        ],
        out_specs=pl.BlockSpec((8, dd), lambda i: (0, 0)),
        out_shape=jax.ShapeDtypeStruct((8, dd), jnp.float32),
        compiler_params=pltpu.CompilerParams(
            dimension_semantics=("arbitrary",)),
    )(x, P, w, b)


def _mlp1_body(xi_ref, p_ref, w1_ref, b1_ref, m_ref, sd_ref, g_ref, be_ref,
               w2_ref, b2_ref, z2_ref, s_ref):
    i = pl.program_id(0)

    @pl.when(i == 0)
    def _():
        s_ref[...] = jnp.zeros_like(s_ref)

    dd = w2_ref.shape[1]
    bm = m_ref[...]
    sd = sd_ref[...]
    g = g_ref[...]
    be = be_ref[...]
    w2 = w2_ref[...]
    b2 = b2_ref[...]
    s1 = jnp.zeros((1, dd), jnp.float32)
    s2 = jnp.zeros((1, dd), jnp.float32)
    for j in range(_K):
        z = _edge1(xi_ref, p_ref, w1_ref, b1_ref, j)
        h = jnp.maximum((z - bm) / sd * g + be, 0.0)
        z2 = jax.lax.dot_general(h, w2, (((1,), (0,)), ((), ())),
                                 preferred_element_type=jnp.float32) + b2
        z2_ref[j] = z2
        s1 = s1 + jnp.sum(z2, axis=0, keepdims=True)
        s2 = s2 + jnp.sum(z2 * z2, axis=0, keepdims=True)
    s_ref[0:1, :] += s1
    s_ref[1:2, :] += s2


def _mlp1(x, P, w1, b1, bm, sd, g, be, w2, b2):
    dd = w2.shape[1]
    return pl.pallas_call(
        _mlp1_body,
        grid=(_NB,),
        in_specs=[
            pl.BlockSpec((_R, 3), lambda i: (i, 0)),
            pl.BlockSpec((_K, _R, 128), lambda i: (0, i, 0)),
            pl.BlockSpec((6, dd), lambda i: (0, 0)),
            pl.BlockSpec((1, dd), lambda i: (0, 0)),
            pl.BlockSpec((1, dd), lambda i: (0, 0)),
            pl.BlockSpec((1, dd), lambda i: (0, 0)),
            pl.BlockSpec((1, dd), lambda i: (0, 0)),
            pl.BlockSpec((1, dd), lambda i: (0, 0)),
            pl.BlockSpec((dd, dd), lambda i: (0, 0)),
            pl.BlockSpec((1, dd), lambda i: (0, 0)),
        ],
        out_specs=[
            pl.BlockSpec((_K, _R, dd), lambda i: (0, i, 0)),
            pl.BlockSpec((8, dd), lambda i: (0, 0)),
        ],
        out_shape=[
            jax.ShapeDtypeStruct((_K, _N, dd), jnp.float32),
            jax.ShapeDtypeStruct((8, dd), jnp.float32),
        ],
        compiler_params=pltpu.CompilerParams(
            dimension_semantics=("arbitrary",)),
    )(x, P, w1, b1, bm, sd, g, be, w2, b2)


def _mlp2max_body(z2_ref, m_ref, sd_ref, g_ref, be_ref, w_ref, b_ref, x1_ref):
    bm = m_ref[...]
    sd = sd_ref[...]
    g = g_ref[...]
    be = be_ref[...]
    w = w_ref[...]
    b = b_ref[...]
    acc = None
    for j in range(_K):
        h = jnp.maximum((z2_ref[j] - bm) / sd * g + be, 0.0)
        z3 = jax.lax.dot_general(h, w, (((1,), (0,)), ((), ())),
                                 preferred_element_type=jnp.float32) + b
        acc = z3 if acc is None else jnp.maximum(acc, z3)
    x1_ref[...] = jnp.concatenate(
        [acc, jnp.zeros((acc.shape[0], 128 - acc.shape[1]), jnp.float32)],
        axis=1)


def _mlp2max(z2, bm, sd, g, be, w, b):
    dd = w.shape[1]
    return pl.pallas_call(
        _mlp2max_body,
        grid=(_NB,),
        in_specs=[
            pl.BlockSpec((_K, _R, dd), lambda i: (0, i, 0)),
            pl.BlockSpec((1, dd), lambda i: (0, 0)),
            pl.BlockSpec((1, dd), lambda i: (0, 0)),
            pl.BlockSpec((1, dd), lambda i: (0, 0)),
            pl.BlockSpec((1, dd), lambda i: (0, 0)),
            pl.BlockSpec((dd, dd), lambda i: (0, 0)),
            pl.BlockSpec((1, dd), lambda i: (0, 0)),
        ],
        out_specs=pl.BlockSpec((_R, 128), lambda i: (i, 0)),
        out_shape=jax.ShapeDtypeStruct((_N, 128), jnp.float32),
        compiler_params=pltpu.CompilerParams(
            dimension_semantics=("arbitrary",)),
    )(z2, bm, sd, g, be, w, b)


def _l1segmax_body(x1_ref, xg_ref, wc_ref, bc_ref, bcol_ref, w_ref, b_ref,
                   seg_ref):
    i = pl.program_id(0)

    @pl.when(i == 0)
    def _():
        seg_ref[...] = jnp.full_like(seg_ref, _NEG)

    xi = x1_ref[...][:, :64]
    wc = wc_ref[...]
    bc = bc_ref[...]
    acc = None
    for j in range(_K):
        e2 = jnp.concatenate([xi, xg_ref[j][:, :64] - xi], axis=1)
        z = jax.lax.dot_general(e2, wc, (((1,), (0,)), ((), ())),
                                preferred_element_type=jnp.float32) + bc
        acc = z if acc is None else jnp.maximum(acc, z)
    e = jnp.concatenate([xi, acc], axis=1)
    o = jax.lax.dot_general(e, w_ref[...], (((1,), (0,)), ((), ())),
                            preferred_element_type=jnp.float32) + b_ref[...]
    bb = bcol_ref[...]
    for s in range(_G):
        vals = jnp.max(jnp.where(bb == s, o, _NEG), axis=0, keepdims=True)
        seg_ref[s:s + 1, :] = jnp.maximum(seg_ref[s:s + 1, :], vals)


def _l1segmax(x1, Xg, wc, bc, batchcol, w, b):
    do = w.shape[1]
    return pl.pallas_call(
        _l1segmax_body,
        grid=(_NB,),
        in_specs=[
            pl.BlockSpec((_R, 128), lambda i: (i, 0)),
            pl.BlockSpec((_K, _R, 128), lambda i: (0, i, 0)),
            pl.BlockSpec((128, 128), lambda i: (0, 0)),
            pl.BlockSpec((1, 128), lambda i: (0, 0)),
            pl.BlockSpec((_R, 1), lambda i: (i, 0)),
            pl.BlockSpec((192, do), lambda i: (0, 0)),
            pl.BlockSpec((1, do), lambda i: (0, 0)),
        ],
        out_specs=pl.BlockSpec((_G, do), lambda i: (0, 0)),
        out_shape=jax.ShapeDtypeStruct((_G, do), jnp.float32),
        compiler_params=pltpu.CompilerParams(
            dimension_semantics=("arbitrary",)),
    )(x1, Xg, wc, bc, batchcol, w, b)


def _head_body(x_ref, w1_ref, b1_ref, w2_ref, b2_ref, w3_ref, b3_ref, o_ref):
    a = jax.lax.dot_general(x_ref[...], w1_ref[...], (((1,), (0,)), ((), ())),
                            preferred_element_type=jnp.float32) + b1_ref[...]
    a = jax.lax.dot_general(a, w2_ref[...], (((1,), (0,)), ((), ())),
                            preferred_element_type=jnp.float32) + b2_ref[...]
    o_ref[...] = jax.lax.dot_general(a, w3_ref[...], (((1,), (0,)), ((), ())),
                                     preferred_element_type=jnp.float32) + b3_ref[...]


def _head(x, w1, b1, w2, b2, w3, b3):
    return pl.pallas_call(
        _head_body,
        out_shape=jax.ShapeDtypeStruct((x.shape[0], w3.shape[1]), jnp.float32),
    )(x, w1, b1, w2, b2, w3, b3)


def kernel(pos, batch, c1_W1, c1_b1, c1_g1, c1_be1, c1_W2, c1_b2, c1_g2,
           c1_be2, c1_W3, c1_b3, c2_W, c2_b, l1_W, l1_b, m1_W, m1_b, m2_W,
           m2_b, h_W, h_b):
    batch = batch.astype(jnp.int32)
    brow = batch.reshape(1, _N)
    bcol = batch.reshape(_N, 1)
    nedges = float(_N * _K)
    seg_lo = jnp.searchsorted(batch, jnp.arange(_G, dtype=batch.dtype),
                              side="left").astype(jnp.int32)
    seg_hi = jnp.searchsorted(batch, jnp.arange(_G, dtype=batch.dtype),
                              side="right").astype(jnp.int32)
    b2d = batch.reshape(_NB, _R)
    blk_start = seg_lo[b2d[:, 0]]
    blk_end = seg_hi[b2d[:, -1]]

    def bn_params(sums):
        m = sums[0] / nedges
        var = sums[1] / nedges - m * m
        sd = jnp.sqrt(var + 1e-5)
        return m.reshape(1, -1), sd.reshape(1, -1)

    idx1 = _knn(pos, brow, bcol, blk_start, blk_end)
    posp = jnp.concatenate([pos, jnp.zeros((_N, 125), jnp.float32)], axis=1)
    P1 = _sc_gather(posp, idx1.reshape(1, -1)).reshape(_K, _N, 128)
    sums1 = _stats(pos, P1, c1_W1, c1_b1.reshape(1, -1))
    m1, sd1 = bn_params(sums1)
    z2, sums2 = _mlp1(pos, P1, c1_W1, c1_b1.reshape(1, -1), m1, sd1,
                      c1_g1.reshape(1, -1), c1_be1.reshape(1, -1),
                      c1_W2, c1_b2.reshape(1, -1))
    m2, sd2 = bn_params(sums2)
    x1 = _mlp2max(z2, m2, sd2, c1_g2.reshape(1, -1), c1_be2.reshape(1, -1),
                  c1_W3, c1_b3.reshape(1, -1))

    idx2 = _knn(x1, brow, bcol, blk_start, blk_end)
    Xg = _sc_gather(x1, idx2.reshape(1, -1)).reshape(_K, _N, 128)

    seg = _l1segmax(x1, Xg, c2_W, c2_b.reshape(1, -1), bcol, l1_W,
                    l1_b.reshape(1, -1))
    return _head(seg, m1_W, m1_b.reshape(1, -1), m2_W, m2_b.reshape(1, -1),
                 h_W, h_b.reshape(1, -1))


# speedup vs baseline: 17.7998x; 1.1346x over previous
import jax
import jax.numpy as jnp
from jax.experimental import pallas as pl
from jax.experimental.pallas import tpu as pltpu
from jax.experimental.pallas import tpu_sc as plsc

_N = 8192
_K = 20
_G = 16
_R = 256
_NB = _N // _R
_NEG = -jnp.inf
_KP = 32
_W_FAST = 1280


def _make_knn_body(window):
    def body(starts_ref, x_ref, xT_ref, brow_ref, bcol_ref, idx_ref):
        i = pl.program_id(0)
        s = starts_ref[i]
        xw = x_ref[pl.ds(s, window), :]
        xtr = xT_ref[...]
        mm = jax.lax.dot_general(xw, xtr, (((1,), (0,)), ((), ())),
                                 preferred_element_type=jnp.float32)
        cn = jnp.sum(xw * xw, axis=1, keepdims=True)
        rn = jnp.sum(xtr * xtr, axis=0, keepdims=True)
        dT = cn + rn - 2.0 * mm
        candio = s + jax.lax.broadcasted_iota(jnp.int32, dT.shape, 0)
        rowio = i * _R + jax.lax.broadcasted_iota(jnp.int32, dT.shape, 1)
        bw = bcol_ref[pl.ds(s, window), :]
        valid = (bw == brow_ref[...]) & (candio != rowio)
        dT = jnp.where(valid, dT, jnp.inf)

        def step(t, d):
            m = jnp.min(d, axis=0, keepdims=True)
            cand = jnp.where(d <= m, candio, jnp.int32(2147483647))
            j = jnp.min(cand, axis=0, keepdims=True)
            idx_ref[pl.ds(t, 1), :] = j
            return jnp.where(candio == j, jnp.inf, d)

        jax.lax.fori_loop(0, _K, step, dT)

    return body


def _knn_call(x, xT, brow, bcol, starts, window):
    n, f = x.shape
    grid_spec = pltpu.PrefetchScalarGridSpec(
        num_scalar_prefetch=1,
        grid=(_NB,),
        in_specs=[
            pl.BlockSpec((n, f), lambda i, *_: (0, 0)),
            pl.BlockSpec((f, _R), lambda i, *_: (0, i)),
            pl.BlockSpec((1, _R), lambda i, *_: (0, i)),
            pl.BlockSpec((n, 1), lambda i, *_: (0, 0)),
        ],
        out_specs=pl.BlockSpec((_KP, _R), lambda i, *_: (0, i)),
    )
    return pl.pallas_call(
        _make_knn_body(window),
        grid_spec=grid_spec,
        out_shape=jax.ShapeDtypeStruct((_KP, n), jnp.int32),
        compiler_params=pltpu.CompilerParams(
            dimension_semantics=("arbitrary",)),
    )(starts, x, xT, brow, bcol)


def _knn(x, batchrow, batchcol, blk_start, blk_end):
    n = x.shape[0]
    xT = x.T
    s_fast = jnp.minimum(blk_start, n - _W_FAST).astype(jnp.int32)
    fits = jnp.all(blk_end - s_fast <= _W_FAST)
    zeros = jnp.zeros_like(s_fast)

    def fast(xx, xxT):
        return _knn_call(xx, xxT, batchrow, batchcol, s_fast, _W_FAST)

    def slow(xx, xxT):
        return _knn_call(xx, xxT, batchrow, batchcol, zeros, n)

    idxp = jax.lax.cond(fits, fast, slow, x, xT)
    return idxp[:_K]


def _sc_gather(values, idx_flat, window=128):
    num_idx = idx_flat.shape[1]
    dv = values.shape[1]
    mesh = plsc.VectorSubcoreMesh(core_axis_name="c", subcore_axis_name="s")

    @pl.kernel(out_type=jax.ShapeDtypeStruct((num_idx, dv), values.dtype),
               mesh=mesh)
    def k(x_hbm, i_hbm, o_hbm):
        def body(i_vmem, o_vmem):
            pltpu.sync_copy(x_hbm.at[i_vmem.at[0]], o_vmem)

        pltpu.emit_pipeline(
            body,
            grid=(num_idx // window,),
            in_specs=[pl.BlockSpec((1, window), index_map=lambda i: (0, i))],
            out_specs=[pl.BlockSpec((window, dv), index_map=lambda i: (i, 0))],
            core_axis_name=("c", "s"),
            dimension_semantics=(pltpu.PARALLEL,),
        )(i_hbm, o_hbm)

    return k(values, idx_flat)


def _edge1(xi_ref, p_ref, w_ref, b_ref, j):
    xi = xi_ref[...]
    xj = p_ref[j][:, :3]
    e = jnp.concatenate([xi, xj - xi], axis=1)
    return jax.lax.dot_general(e, w_ref[...], (((1,), (0,)), ((), ())),
                               preferred_element_type=jnp.float32) + b_ref[...]


def _stats_body(xi_ref, p_ref, w_ref, b_ref, s_ref):
    i = pl.program_id(0)

    @pl.when(i == 0)
    def _():
        s_ref[...] = jnp.zeros_like(s_ref)

    dd = w_ref.shape[1]
    s1 = jnp.zeros((1, dd), jnp.float32)
    s2 = jnp.zeros((1, dd), jnp.float32)
    for j in range(_K):
        z = _edge1(xi_ref, p_ref, w_ref, b_ref, j)
        s1 = s1 + jnp.sum(z, axis=0, keepdims=True)
        s2 = s2 + jnp.sum(z * z, axis=0, keepdims=True)
    s_ref[0:1, :] += s1
    s_ref[1:2, :] += s2


def _stats(x, P, w, b):
    dd = w.shape[1]
    return pl.pallas_call(
        _stats_body,
        grid=(_NB,),
        in_specs=[
            pl.BlockSpec((_R, 3), lambda i: (i, 0)),
            pl.BlockSpec((_K, _R, 128), lambda i: (0, i, 0)),
            pl.BlockSpec((6, dd), lambda i: (0, 0)),
            pl.BlockSpec((1, dd), lambda i: (0, 0)),
        ],
        out_specs=pl.BlockSpec((8, dd), lambda i: (0, 0)),
        out_shape=jax.ShapeDtypeStruct((8, dd), jnp.float32),
        compiler_params=pltpu.CompilerParams(
            dimension_semantics=("arbitrary",)),
    )(x, P, w, b)


def _mlp1_body(xi_ref, p_ref, w1_ref, b1_ref, m_ref, sd_ref, g_ref, be_ref,
               w2_ref, b2_ref, z2_ref, s_ref):
    i = pl.program_id(0)

    @pl.when(i == 0)
    def _():
        s_ref[...] = jnp.zeros_like(s_ref)

    dd = w2_ref.shape[1]
    bm = m_ref[...]
    sd = sd_ref[...]
    g = g_ref[...]
    be = be_ref[...]
    w2 = w2_ref[...]
    b2 = b2_ref[...]
    s1 = jnp.zeros((1, dd), jnp.float32)
    s2 = jnp.zeros((1, dd), jnp.float32)
    for j in range(_K):
        z = _edge1(xi_ref, p_ref, w1_ref, b1_ref, j)
        h = jnp.maximum((z - bm) / sd * g + be, 0.0)
        z2 = jax.lax.dot_general(h, w2, (((1,), (0,)), ((), ())),
                                 preferred_element_type=jnp.float32) + b2
        z2_ref[j] = z2
        s1 = s1 + jnp.sum(z2, axis=0, keepdims=True)
        s2 = s2 + jnp.sum(z2 * z2, axis=0, keepdims=True)
    s_ref[0:1, :] += s1
    s_ref[1:2, :] += s2


def _mlp1(x, P, w1, b1, bm, sd, g, be, w2, b2):
    dd = w2.shape[1]
    return pl.pallas_call(
        _mlp1_body,
        grid=(_NB,),
        in_specs=[
            pl.BlockSpec((_R, 3), lambda i: (i, 0)),
            pl.BlockSpec((_K, _R, 128), lambda i: (0, i, 0)),
            pl.BlockSpec((6, dd), lambda i: (0, 0)),
            pl.BlockSpec((1, dd), lambda i: (0, 0)),
            pl.BlockSpec((1, dd), lambda i: (0, 0)),
            pl.BlockSpec((1, dd), lambda i: (0, 0)),
            pl.BlockSpec((1, dd), lambda i: (0, 0)),
            pl.BlockSpec((1, dd), lambda i: (0, 0)),
            pl.BlockSpec((dd, dd), lambda i: (0, 0)),
            pl.BlockSpec((1, dd), lambda i: (0, 0)),
        ],
        out_specs=[
            pl.BlockSpec((_K, _R, dd), lambda i: (0, i, 0)),
            pl.BlockSpec((8, dd), lambda i: (0, 0)),
        ],
        out_shape=[
            jax.ShapeDtypeStruct((_K, _N, dd), jnp.float32),
            jax.ShapeDtypeStruct((8, dd), jnp.float32),
        ],
        compiler_params=pltpu.CompilerParams(
            dimension_semantics=("arbitrary",)),
    )(x, P, w1, b1, bm, sd, g, be, w2, b2)


def _mlp2max_body(z2_ref, m_ref, sd_ref, g_ref, be_ref, w_ref, b_ref, x1_ref):
    bm = m_ref[...]
    sd = sd_ref[...]
    g = g_ref[...]
    be = be_ref[...]
    w = w_ref[...]
    b = b_ref[...]
    acc = None
    for j in range(_K):
        h = jnp.maximum((z2_ref[j] - bm) / sd * g + be, 0.0)
        z3 = jax.lax.dot_general(h, w, (((1,), (0,)), ((), ())),
                                 preferred_element_type=jnp.float32) + b
        acc = z3 if acc is None else jnp.maximum(acc, z3)
    x1_ref[...] = jnp.concatenate(
        [acc, jnp.zeros((acc.shape[0], 128 - acc.shape[1]), jnp.float32)],
        axis=1)


def _mlp2max(z2, bm, sd, g, be, w, b):
    dd = w.shape[1]
    return pl.pallas_call(
        _mlp2max_body,
        grid=(_NB,),
        in_specs=[
            pl.BlockSpec((_K, _R, dd), lambda i: (0, i, 0)),
            pl.BlockSpec((1, dd), lambda i: (0, 0)),
            pl.BlockSpec((1, dd), lambda i: (0, 0)),
            pl.BlockSpec((1, dd), lambda i: (0, 0)),
            pl.BlockSpec((1, dd), lambda i: (0, 0)),
            pl.BlockSpec((dd, dd), lambda i: (0, 0)),
            pl.BlockSpec((1, dd), lambda i: (0, 0)),
        ],
        out_specs=pl.BlockSpec((_R, 128), lambda i: (i, 0)),
        out_shape=jax.ShapeDtypeStruct((_N, 128), jnp.float32),
        compiler_params=pltpu.CompilerParams(
            dimension_semantics=("arbitrary",)),
    )(z2, bm, sd, g, be, w, b)


def _l1segmax_body(x1_ref, xg_ref, wc_ref, bc_ref, bcol_ref, w_ref, b_ref,
                   seg_ref):
    i = pl.program_id(0)

    @pl.when(i == 0)
    def _():
        seg_ref[...] = jnp.full_like(seg_ref, _NEG)

    xi = x1_ref[...][:, :64]
    wc = wc_ref[...]
    bc = bc_ref[...]
    acc = None
    for j in range(_K):
        e2 = jnp.concatenate([xi, xg_ref[j][:, :64] - xi], axis=1)
        z = jax.lax.dot_general(e2, wc, (((1,), (0,)), ((), ())),
                                preferred_element_type=jnp.float32) + bc
        acc = z if acc is None else jnp.maximum(acc, z)
    e = jnp.concatenate([xi, acc], axis=1)
    o = jax.lax.dot_general(e, w_ref[...], (((1,), (0,)), ((), ())),
                            preferred_element_type=jnp.float32) + b_ref[...]
    bb = bcol_ref[...]
    for s in range(_G):
        vals = jnp.max(jnp.where(bb == s, o, _NEG), axis=0, keepdims=True)
        seg_ref[s:s + 1, :] = jnp.maximum(seg_ref[s:s + 1, :], vals)


def _l1segmax(x1, Xg, wc, bc, batchcol, w, b):
    do = w.shape[1]
    return pl.pallas_call(
        _l1segmax_body,
        grid=(_NB,),
        in_specs=[
            pl.BlockSpec((_R, 128), lambda i: (i, 0)),
            pl.BlockSpec((_K, _R, 128), lambda i: (0, i, 0)),
            pl.BlockSpec((128, 128), lambda i: (0, 0)),
            pl.BlockSpec((1, 128), lambda i: (0, 0)),
            pl.BlockSpec((_R, 1), lambda i: (i, 0)),
            pl.BlockSpec((192, do), lambda i: (0, 0)),
            pl.BlockSpec((1, do), lambda i: (0, 0)),
        ],
        out_specs=pl.BlockSpec((_G, do), lambda i: (0, 0)),
        out_shape=jax.ShapeDtypeStruct((_G, do), jnp.float32),
        compiler_params=pltpu.CompilerParams(
            dimension_semantics=("arbitrary",)),
    )(x1, Xg, wc, bc, batchcol, w, b)


def _head_body(x_ref, w1_ref, b1_ref, w2_ref, b2_ref, w3_ref, b3_ref, o_ref):
    a = jax.lax.dot_general(x_ref[...], w1_ref[...], (((1,), (0,)), ((), ())),
                            preferred_element_type=jnp.float32) + b1_ref[...]
    a = jax.lax.dot_general(a, w2_ref[...], (((1,), (0,)), ((), ())),
                            preferred_element_type=jnp.float32) + b2_ref[...]
    o_ref[...] = jax.lax.dot_general(a, w3_ref[...], (((1,), (0,)), ((), ())),
                                     preferred_element_type=jnp.float32) + b3_ref[...]


def _head(x, w1, b1, w2, b2, w3, b3):
    return pl.pallas_call(
        _head_body,
        out_shape=jax.ShapeDtypeStruct((x.shape[0], w3.shape[1]), jnp.float32),
    )(x, w1, b1, w2, b2, w3, b3)


def kernel(pos, batch, c1_W1, c1_b1, c1_g1, c1_be1, c1_W2, c1_b2, c1_g2,
           c1_be2, c1_W3, c1_b3, c2_W, c2_b, l1_W, l1_b, m1_W, m1_b, m2_W,
           m2_b, h_W, h_b):
    batch = batch.astype(jnp.int32)
    brow = batch.reshape(1, _N)
    bcol = batch.reshape(_N, 1)
    nedges = float(_N * _K)
    seg_lo = jnp.searchsorted(batch, jnp.arange(_G, dtype=batch.dtype),
                              side="left").astype(jnp.int32)
    seg_hi = jnp.searchsorted(batch, jnp.arange(_G, dtype=batch.dtype),
                              side="right").astype(jnp.int32)
    b2d = batch.reshape(_NB, _R)
    blk_start = seg_lo[b2d[:, 0]]
    blk_end = seg_hi[b2d[:, -1]]

    def bn_params(sums):
        m = sums[0] / nedges
        var = sums[1] / nedges - m * m
        sd = jnp.sqrt(var + 1e-5)
        return m.reshape(1, -1), sd.reshape(1, -1)

    idx1 = _knn(pos, brow, bcol, blk_start, blk_end)
    posp = jnp.concatenate([pos, jnp.zeros((_N, 125), jnp.float32)], axis=1)
    P1 = _sc_gather(posp, idx1.reshape(1, -1)).reshape(_K, _N, 128)
    sums1 = _stats(pos, P1, c1_W1, c1_b1.reshape(1, -1))
    m1, sd1 = bn_params(sums1)
    z2, sums2 = _mlp1(pos, P1, c1_W1, c1_b1.reshape(1, -1), m1, sd1,
                      c1_g1.reshape(1, -1), c1_be1.reshape(1, -1),
                      c1_W2, c1_b2.reshape(1, -1))
    m2, sd2 = bn_params(sums2)
    x1 = _mlp2max(z2, m2, sd2, c1_g2.reshape(1, -1), c1_be2.reshape(1, -1),
                  c1_W3, c1_b3.reshape(1, -1))

    idx2 = _knn(x1, brow, bcol, blk_start, blk_end)
    Xg = _sc_gather(x1, idx2.reshape(1, -1)).reshape(_K, _N, 128)

    seg = _l1segmax(x1, Xg, c2_W, c2_b.reshape(1, -1), bcol, l1_W,
                    l1_b.reshape(1, -1))
    return _head(seg, m1_W, m1_b.reshape(1, -1), m2_W, m2_b.reshape(1, -1),
                 h_W, h_b.reshape(1, -1))
